# baseline (speedup 1.0000x reference)
import functools

import jax
import jax.numpy as jnp
from jax import lax
from jax.experimental import pallas as pl
from jax.experimental.pallas import tpu as pltpu
from jax.experimental.pallas import tpu_sc as plsc

D = 1024
E = 8
DFF = 4096
NT = 2048
TILE = 512
T = NT // TILE + (E - 1)
P = T * TILE
FB = 1024
NFF = DFF // FB
NW = 32
TPW = NT // NW


def _gate_body(x_ref, gw_ref, gb_ref,
               pos_ref, mv_ref, g_ref, v_ref, jm_ref, loss_ref, oh_ref):
    x = x_ref[...]
    logits = jnp.dot(x, gw_ref[...], preferred_element_type=jnp.float32)
    logits = logits + gb_ref[...]
    lmax = jnp.max(logits, axis=1, keepdims=True)
    ex = jnp.exp(logits - lmax)
    den = jnp.sum(ex, axis=1, keepdims=True)
    mv_ref[...] = 1.0 / den

    idx8 = lax.broadcasted_iota(jnp.int32, (NT, E), 1)
    top1 = jnp.min(jnp.where(logits == lmax, idx8, E), axis=1, keepdims=True)
    oh = (idx8 == top1).astype(jnp.float32)
    oh_ref[...] = oh

    counts = jnp.sum(oh, axis=0, keepdims=True)
    rc = ((counts.astype(jnp.int32) + TILE - 1) // TILE) * TILE
    rcf = rc.astype(jnp.float32)
    jrow = lax.broadcasted_iota(jnp.int32, (E, E), 0)
    ecol = lax.broadcasted_iota(jnp.int32, (E, E), 1)
    upper = (jrow < ecol).astype(jnp.float32)
    starts = jnp.dot(rcf, upper, preferred_element_type=jnp.float32)
    tcount = jnp.sum(rcf)

    tpos = (lax.broadcasted_iota(jnp.int32, (128, E), 0) * TILE).astype(
        jnp.float32)
    q = jnp.minimum(tpos, tcount - 1.0)
    ge = (starts <= q).astype(jnp.int32)
    g_ref[...] = jnp.sum(ge, axis=1, keepdims=True) - 1
    v_ref[...] = (tpos[:, :1] < tcount).astype(jnp.int32)

    nv = (tcount / TILE).astype(jnp.int32)
    l = lax.broadcasted_iota(jnp.int32, (128, 1), 0)
    ti = l // NFF
    tj = l % NFF
    serp = jnp.where(ti % 2 == 0, tj, NFF - 1 - tj)
    last_chunk = jnp.where((nv - 1) % 2 == 0, NFF - 1, 0)
    jm = jnp.where(ti < nv, serp, last_chunk)
    jm_ref[...] = jnp.where(l == 127, nv - 1, jm)

    loss_ref[...] = jnp.sum(counts, axis=1, keepdims=True) * (0.01 / NT)

    rr = lax.broadcasted_iota(jnp.int32, (128, 128), 0)
    cc = lax.broadcasted_iota(jnp.int32, (128, 128), 1)
    ltri = (cc < rr).astype(jnp.float32)

    def chunk(k, carry):
        ohc = oh_ref[pl.ds(k * 128, 128), :]
        rank = jnp.dot(ltri, ohc, preferred_element_type=jnp.float32) + carry
        posf = jnp.sum(ohc * (starts + rank), axis=1, keepdims=True)
        pos_ref[pl.ds(k * 128, 128), :] = posf.astype(jnp.int32)
        return carry + jnp.sum(ohc, axis=0, keepdims=True)

    lax.fori_loop(0, NT // 128, chunk, jnp.zeros((1, E), jnp.float32))


_gate_call = pl.pallas_call(
    _gate_body,
    out_shape=(
        jax.ShapeDtypeStruct((NT, 1), jnp.int32),
        jax.ShapeDtypeStruct((NT, 1), jnp.float32),
        jax.ShapeDtypeStruct((128, 1), jnp.int32),
        jax.ShapeDtypeStruct((128, 1), jnp.int32),
        jax.ShapeDtypeStruct((128, 1), jnp.int32),
        jax.ShapeDtypeStruct((1, 1), jnp.float32),
    ),
    scratch_shapes=[pltpu.VMEM((NT, E), jnp.float32)],
)


@functools.cache
def _make_dispatch():
    mesh = plsc.VectorSubcoreMesh(core_axis_name="c", subcore_axis_name="s")

    H = TPW // 2

    @functools.partial(
        pl.kernel,
        out_type=(
            jax.ShapeDtypeStruct((P, D), jnp.float32),
            jax.ShapeDtypeStruct((P, 128), jnp.float32),
        ),
        mesh=mesh,
        scratch_types=[
            pltpu.VMEM((H,), jnp.int32),
            pltpu.VMEM((H,), jnp.int32),
            pltpu.VMEM((TPW,), jnp.int32),
            pltpu.VMEM((H, D), jnp.float32),
            pltpu.VMEM((H, D), jnp.float32),
            pltpu.VMEM((TPW,), jnp.float32),
            pltpu.VMEM((TPW, 128), jnp.float32),
            pltpu.SemaphoreType.DMA,
            pltpu.SemaphoreType.DMA,
            pltpu.SemaphoreType.DMA,
            pltpu.SemaphoreType.DMA,
            pltpu.SemaphoreType.DMA,
        ],
    )
    def dispatch(x_hbm, pos_hbm, mv_hbm, xs_hbm, mv2_hbm,
                 pos_a, pos_b, pos_v, rows_a, rows_b, mv_v, mvr_v,
                 s1, s2, s3, s4, s5):
        wid = lax.axis_index("s") * 2 + lax.axis_index("c")
        base = wid * TPW
        in_a = pltpu.async_copy(x_hbm.at[pl.ds(base, H)], rows_a, s1)
        in_b = pltpu.async_copy(x_hbm.at[pl.ds(base + H, H)], rows_b, s2)
        pltpu.sync_copy(pos_hbm.at[pl.ds(base, H)], pos_a)
        pltpu.sync_copy(pos_hbm.at[pl.ds(base + H, H)], pos_b)
        pltpu.sync_copy(pos_hbm.at[pl.ds(base, TPW)], pos_v)
        pltpu.sync_copy(mv_hbm.at[pl.ds(base, TPW)], mv_v)
        for c in range(TPW // 16):
            v = mv_v[pl.ds(c * 16, 16)]
            for l in range(16):
                mvr_v[c * 16 + l, pl.ds(0, 16)] = jnp.full((16,), v[l],
                                                           jnp.float32)
        cp_mv = pltpu.async_copy(mvr_v, mv2_hbm.at[pos_v], s5)
        in_a.wait()
        out_a = pltpu.async_copy(rows_a, xs_hbm.at[pos_a], s3)
        in_b.wait()
        out_b = pltpu.async_copy(rows_b, xs_hbm.at[pos_b], s4)
        out_a.wait()
        out_b.wait()
        cp_mv.wait()

    return dispatch


def _ffn_body(g_s, v_s, jm_s, xs_ref, w1_ref, b1_ref, w2_ref, b2_ref,
              mvt_ref, out_ref):
    i = pl.program_id(0)
    j = pl.program_id(1)

    @pl.when(v_s[i] == 1)
    def _():
        xb = xs_ref[...].astype(jnp.bfloat16)
        h = jnp.dot(xb, w1_ref[0].astype(jnp.bfloat16),
                    preferred_element_type=jnp.float32)
        hb = jnp.maximum(h + b1_ref[0], 0.0).astype(jnp.bfloat16)
        part = jnp.dot(hb, w2_ref[0].astype(jnp.bfloat16),
                       preferred_element_type=jnp.float32)

        @pl.when(j == 0)
        def _():
            out_ref[...] = part

        @pl.when(jnp.logical_and(j > 0, j < NFF - 1))
        def _():
            out_ref[...] += part

        @pl.when(j == NFF - 1)
        def _():
            out_ref[...] = (out_ref[...] + part + b2_ref[0]) \
                * mvt_ref[...][:, :1]


def _ti(i, v, jm):
    return jnp.where(v[i] == 1, i, jm[127])


_ffn_call = pl.pallas_call(
    _ffn_body,
    grid_spec=pltpu.PrefetchScalarGridSpec(
        num_scalar_prefetch=3,
        grid=(T, NFF),
        in_specs=[
            pl.BlockSpec((TILE, D), lambda i, j, g, v, jm: (_ti(i, v, jm), 0)),
            pl.BlockSpec((1, D, FB),
                         lambda i, j, g, v, jm: (g[i], 0, jm[NFF * i + j])),
            pl.BlockSpec((1, 1, FB),
                         lambda i, j, g, v, jm: (g[i], 0, jm[NFF * i + j])),
            pl.BlockSpec((1, FB, D),
                         lambda i, j, g, v, jm: (g[i], jm[NFF * i + j], 0)),
            pl.BlockSpec((1, 1, D), lambda i, j, g, v, jm: (g[i], 0, 0)),
            pl.BlockSpec((TILE, 128),
                         lambda i, j, g, v, jm: (_ti(i, v, jm), 0)),
        ],
        out_specs=pl.BlockSpec((TILE, D),
                               lambda i, j, g, v, jm: (_ti(i, v, jm), 0)),
    ),
    out_shape=jax.ShapeDtypeStruct((P, D), jnp.float32),
    compiler_params=pltpu.CompilerParams(
        dimension_semantics=("arbitrary", "arbitrary")),
)


@functools.cache
def _make_combine():
    mesh = plsc.VectorSubcoreMesh(core_axis_name="c", subcore_axis_name="s")

    H = TPW // 2

    @functools.partial(
        pl.kernel,
        out_type=jax.ShapeDtypeStruct((NT, D), jnp.float32),
        mesh=mesh,
        scratch_types=[
            pltpu.VMEM((H,), jnp.int32),
            pltpu.VMEM((H,), jnp.int32),
            pltpu.VMEM((H, D), jnp.float32),
            pltpu.VMEM((H, D), jnp.float32),
            pltpu.SemaphoreType.DMA,
            pltpu.SemaphoreType.DMA,
            pltpu.SemaphoreType.DMA,
            pltpu.SemaphoreType.DMA,
        ],
    )
    def combine(ys_hbm, pos_hbm, out_hbm,
                pos_a, pos_b, rows_a, rows_b, s1, s2, s3, s4):
        wid = lax.axis_index("s") * 2 + lax.axis_index("c")
        base = wid * TPW
        pltpu.sync_copy(pos_hbm.at[pl.ds(base, H)], pos_a)
        pltpu.sync_copy(pos_hbm.at[pl.ds(base + H, H)], pos_b)
        in_a = pltpu.async_copy(ys_hbm.at[pos_a], rows_a, s1)
        in_b = pltpu.async_copy(ys_hbm.at[pos_b], rows_b, s2)
        in_a.wait()
        out_a = pltpu.async_copy(rows_a, out_hbm.at[pl.ds(base, H)], s3)
        in_b.wait()
        out_b = pltpu.async_copy(rows_b, out_hbm.at[pl.ds(base + H, H)], s4)
        out_a.wait()
        out_b.wait()

    return combine


def kernel(x, gate_w, gate_b, w1, b1, w2, b2):
    x2d = x.reshape(NT, D)
    pos2, mv2, g2, v2, jm2, loss = _gate_call(x2d, gate_w,
                                              gate_b.reshape(1, E))
    pos = pos2.reshape(NT)
    mv = mv2.reshape(NT)
    g = g2.reshape(128)[:T]
    vld = v2.reshape(128)[:T]
    jm = jm2.reshape(128)
    xs, mvt = _make_dispatch()(x2d, pos, mv)
    ys = _ffn_call(g, vld, jm, xs, w1, b1.reshape(E, 1, DFF), w2,
                   b2.reshape(E, 1, D), mvt)
    y2d = _make_combine()(ys, pos)
    return y2d.reshape(x.shape), loss.reshape(())


# speedup vs baseline: 1.1136x; 1.1136x over previous
import functools

import jax
import jax.numpy as jnp
from jax import lax
from jax.experimental import pallas as pl
from jax.experimental.pallas import tpu as pltpu
from jax.experimental.pallas import tpu_sc as plsc

D = 1024
E = 8
DFF = 4096
NT = 2048
TILE = 384
T = -(-NT // TILE) + (E - 1)
P = T * TILE
FB = 2048
NFF = DFF // FB
NW = 32
TPW = NT // NW


def _gate_body(x_ref, gw_ref, gb_ref,
               pos_ref, mv_ref, g_ref, v_ref, jm_ref, loss_ref, oh_ref):
    x = x_ref[...]
    logits = jnp.dot(x, gw_ref[...], preferred_element_type=jnp.float32)
    logits = logits + gb_ref[...]
    lmax = jnp.max(logits, axis=1, keepdims=True)
    ex = jnp.exp(logits - lmax)
    den = jnp.sum(ex, axis=1, keepdims=True)
    mv_ref[...] = 1.0 / den

    idx8 = lax.broadcasted_iota(jnp.int32, (NT, E), 1)
    top1 = jnp.min(jnp.where(logits == lmax, idx8, E), axis=1, keepdims=True)
    oh = (idx8 == top1).astype(jnp.float32)
    oh_ref[...] = oh

    counts = jnp.sum(oh, axis=0, keepdims=True)
    rc = ((counts.astype(jnp.int32) + TILE - 1) // TILE) * TILE
    rcf = rc.astype(jnp.float32)
    jrow = lax.broadcasted_iota(jnp.int32, (E, E), 0)
    ecol = lax.broadcasted_iota(jnp.int32, (E, E), 1)
    upper = (jrow < ecol).astype(jnp.float32)
    starts = jnp.dot(rcf, upper, preferred_element_type=jnp.float32)
    tcount = jnp.sum(rcf)

    tpos = (lax.broadcasted_iota(jnp.int32, (128, E), 0) * TILE).astype(
        jnp.float32)
    q = jnp.minimum(tpos, tcount - 1.0)
    ge = (starts <= q).astype(jnp.int32)
    g_ref[...] = jnp.sum(ge, axis=1, keepdims=True) - 1
    v_ref[...] = (tpos[:, :1] < tcount).astype(jnp.int32)

    nv = (tcount / TILE).astype(jnp.int32)
    l = lax.broadcasted_iota(jnp.int32, (128, 1), 0)
    ti = l // NFF
    tj = l % NFF
    serp = jnp.where(ti % 2 == 0, tj, NFF - 1 - tj)
    last_chunk = jnp.where((nv - 1) % 2 == 0, NFF - 1, 0)
    jm = jnp.where(ti < nv, serp, last_chunk)
    jm_ref[...] = jnp.where(l == 127, nv - 1, jm)

    loss_ref[...] = jnp.sum(counts, axis=1, keepdims=True) * (0.01 / NT)

    rr = lax.broadcasted_iota(jnp.int32, (128, 128), 0)
    cc = lax.broadcasted_iota(jnp.int32, (128, 128), 1)
    ltri = (cc < rr).astype(jnp.float32)

    def chunk(k, carry):
        ohc = oh_ref[pl.ds(k * 128, 128), :]
        rank = jnp.dot(ltri, ohc, preferred_element_type=jnp.float32) + carry
        posf = jnp.sum(ohc * (starts + rank), axis=1, keepdims=True)
        pos_ref[pl.ds(k * 128, 128), :] = posf.astype(jnp.int32)
        return carry + jnp.sum(ohc, axis=0, keepdims=True)

    lax.fori_loop(0, NT // 128, chunk, jnp.zeros((1, E), jnp.float32))


_gate_call = pl.pallas_call(
    _gate_body,
    out_shape=(
        jax.ShapeDtypeStruct((NT, 1), jnp.int32),
        jax.ShapeDtypeStruct((NT, 1), jnp.float32),
        jax.ShapeDtypeStruct((128, 1), jnp.int32),
        jax.ShapeDtypeStruct((128, 1), jnp.int32),
        jax.ShapeDtypeStruct((128, 1), jnp.int32),
        jax.ShapeDtypeStruct((1, 1), jnp.float32),
    ),
    scratch_shapes=[pltpu.VMEM((NT, E), jnp.float32)],
)


@functools.cache
def _make_dispatch():
    mesh = plsc.VectorSubcoreMesh(core_axis_name="c", subcore_axis_name="s")

    H = TPW // 2

    @functools.partial(
        pl.kernel,
        out_type=(
            jax.ShapeDtypeStruct((P, D), jnp.float32),
            jax.ShapeDtypeStruct((P, 128), jnp.float32),
        ),
        mesh=mesh,
        scratch_types=[
            pltpu.VMEM((H,), jnp.int32),
            pltpu.VMEM((H,), jnp.int32),
            pltpu.VMEM((TPW,), jnp.int32),
            pltpu.VMEM((H, D), jnp.float32),
            pltpu.VMEM((H, D), jnp.float32),
            pltpu.VMEM((TPW,), jnp.float32),
            pltpu.VMEM((TPW, 128), jnp.float32),
            pltpu.SemaphoreType.DMA,
            pltpu.SemaphoreType.DMA,
            pltpu.SemaphoreType.DMA,
            pltpu.SemaphoreType.DMA,
            pltpu.SemaphoreType.DMA,
        ],
    )
    def dispatch(x_hbm, pos_hbm, mv_hbm, xs_hbm, mv2_hbm,
                 pos_a, pos_b, pos_v, rows_a, rows_b, mv_v, mvr_v,
                 s1, s2, s3, s4, s5):
        wid = lax.axis_index("s") * 2 + lax.axis_index("c")
        base = wid * TPW
        in_a = pltpu.async_copy(x_hbm.at[pl.ds(base, H)], rows_a, s1)
        in_b = pltpu.async_copy(x_hbm.at[pl.ds(base + H, H)], rows_b, s2)
        pltpu.sync_copy(pos_hbm.at[pl.ds(base, H)], pos_a)
        pltpu.sync_copy(pos_hbm.at[pl.ds(base + H, H)], pos_b)
        pltpu.sync_copy(pos_hbm.at[pl.ds(base, TPW)], pos_v)
        pltpu.sync_copy(mv_hbm.at[pl.ds(base, TPW)], mv_v)
        for c in range(TPW // 16):
            v = mv_v[pl.ds(c * 16, 16)]
            for l in range(16):
                mvr_v[c * 16 + l, pl.ds(0, 16)] = jnp.full((16,), v[l],
                                                           jnp.float32)
        cp_mv = pltpu.async_copy(mvr_v, mv2_hbm.at[pos_v], s5)
        in_a.wait()
        out_a = pltpu.async_copy(rows_a, xs_hbm.at[pos_a], s3)
        in_b.wait()
        out_b = pltpu.async_copy(rows_b, xs_hbm.at[pos_b], s4)
        out_a.wait()
        out_b.wait()
        cp_mv.wait()

    return dispatch


def _ffn_body(g_s, v_s, jm_s, xs_ref, w1_ref, b1_ref, w2_ref, b2_ref,
              mvt_ref, out_ref):
    i = pl.program_id(0)
    j = pl.program_id(1)

    @pl.when(v_s[i] == 1)
    def _():
        xb = xs_ref[...].astype(jnp.bfloat16)
        h = jnp.dot(xb, w1_ref[0].astype(jnp.bfloat16),
                    preferred_element_type=jnp.float32)
        hb = jnp.maximum(h + b1_ref[0], 0.0).astype(jnp.bfloat16)
        part = jnp.dot(hb, w2_ref[0].astype(jnp.bfloat16),
                       preferred_element_type=jnp.float32)

        @pl.when(j == 0)
        def _():
            out_ref[...] = part

        @pl.when(jnp.logical_and(j > 0, j < NFF - 1))
        def _():
            out_ref[...] += part

        @pl.when(j == NFF - 1)
        def _():
            out_ref[...] = (out_ref[...] + part + b2_ref[0]) \
                * mvt_ref[...][:, :1]


def _ti(i, v, jm):
    return jnp.where(v[i] == 1, i, jm[127])


_ffn_call = pl.pallas_call(
    _ffn_body,
    grid_spec=pltpu.PrefetchScalarGridSpec(
        num_scalar_prefetch=3,
        grid=(T, NFF),
        in_specs=[
            pl.BlockSpec((TILE, D), lambda i, j, g, v, jm: (_ti(i, v, jm), 0)),
            pl.BlockSpec((1, D, FB),
                         lambda i, j, g, v, jm: (g[i], 0, jm[NFF * i + j])),
            pl.BlockSpec((1, 1, FB),
                         lambda i, j, g, v, jm: (g[i], 0, jm[NFF * i + j])),
            pl.BlockSpec((1, FB, D),
                         lambda i, j, g, v, jm: (g[i], jm[NFF * i + j], 0)),
            pl.BlockSpec((1, 1, D), lambda i, j, g, v, jm: (g[i], 0, 0)),
            pl.BlockSpec((TILE, 128),
                         lambda i, j, g, v, jm: (_ti(i, v, jm), 0)),
        ],
        out_specs=pl.BlockSpec((TILE, D),
                               lambda i, j, g, v, jm: (_ti(i, v, jm), 0)),
    ),
    out_shape=jax.ShapeDtypeStruct((P, D), jnp.float32),
    compiler_params=pltpu.CompilerParams(
        dimension_semantics=("arbitrary", "arbitrary")),
)


@functools.cache
def _make_combine():
    mesh = plsc.VectorSubcoreMesh(core_axis_name="c", subcore_axis_name="s")

    H = TPW // 2

    @functools.partial(
        pl.kernel,
        out_type=jax.ShapeDtypeStruct((NT, D), jnp.float32),
        mesh=mesh,
        scratch_types=[
            pltpu.VMEM((H,), jnp.int32),
            pltpu.VMEM((H,), jnp.int32),
            pltpu.VMEM((H, D), jnp.float32),
            pltpu.VMEM((H, D), jnp.float32),
            pltpu.SemaphoreType.DMA,
            pltpu.SemaphoreType.DMA,
            pltpu.SemaphoreType.DMA,
            pltpu.SemaphoreType.DMA,
        ],
    )
    def combine(ys_hbm, pos_hbm, out_hbm,
                pos_a, pos_b, rows_a, rows_b, s1, s2, s3, s4):
        wid = lax.axis_index("s") * 2 + lax.axis_index("c")
        base = wid * TPW
        pltpu.sync_copy(pos_hbm.at[pl.ds(base, H)], pos_a)
        pltpu.sync_copy(pos_hbm.at[pl.ds(base + H, H)], pos_b)
        in_a = pltpu.async_copy(ys_hbm.at[pos_a], rows_a, s1)
        in_b = pltpu.async_copy(ys_hbm.at[pos_b], rows_b, s2)
        in_a.wait()
        out_a = pltpu.async_copy(rows_a, out_hbm.at[pl.ds(base, H)], s3)
        in_b.wait()
        out_b = pltpu.async_copy(rows_b, out_hbm.at[pl.ds(base + H, H)], s4)
        out_a.wait()
        out_b.wait()

    return combine


def kernel(x, gate_w, gate_b, w1, b1, w2, b2):
    x2d = x.reshape(NT, D)
    pos2, mv2, g2, v2, jm2, loss = _gate_call(x2d, gate_w,
                                              gate_b.reshape(1, E))
    pos = pos2.reshape(NT)
    mv = mv2.reshape(NT)
    g = g2.reshape(128)[:T]
    vld = v2.reshape(128)[:T]
    jm = jm2.reshape(128)
    xs, mvt = _make_dispatch()(x2d, pos, mv)
    ys = _ffn_call(g, vld, jm, xs, w1, b1.reshape(E, 1, DFF), w2,
                   b2.reshape(E, 1, D), mvt)
    y2d = _make_combine()(ys, pos)
    return y2d.reshape(x.shape), loss.reshape(())


# speedup vs baseline: 1.1287x; 1.0136x over previous
import functools

import jax
import jax.numpy as jnp
from jax import lax
from jax.experimental import pallas as pl
from jax.experimental.pallas import tpu as pltpu
from jax.experimental.pallas import tpu_sc as plsc

D = 1024
E = 8
DFF = 4096
NT = 2048
TILE = 320
T = -(-NT // TILE) + (E - 1)
P = T * TILE
FB = 2048
NFF = DFF // FB
NW = 32
TPW = NT // NW


def _gate_body(x_ref, gw_ref, gb_ref,
               pos_ref, mv_ref, g_ref, v_ref, jm_ref, loss_ref, oh_ref):
    x = x_ref[...]
    logits = jnp.dot(x, gw_ref[...], preferred_element_type=jnp.float32)
    logits = logits + gb_ref[...]
    lmax = jnp.max(logits, axis=1, keepdims=True)
    ex = jnp.exp(logits - lmax)
    den = jnp.sum(ex, axis=1, keepdims=True)
    mv_ref[...] = 1.0 / den

    idx8 = lax.broadcasted_iota(jnp.int32, (NT, E), 1)
    top1 = jnp.min(jnp.where(logits == lmax, idx8, E), axis=1, keepdims=True)
    oh = (idx8 == top1).astype(jnp.float32)
    oh_ref[...] = oh

    counts = jnp.sum(oh, axis=0, keepdims=True)
    rc = ((counts.astype(jnp.int32) + TILE - 1) // TILE) * TILE
    rcf = rc.astype(jnp.float32)
    jrow = lax.broadcasted_iota(jnp.int32, (E, E), 0)
    ecol = lax.broadcasted_iota(jnp.int32, (E, E), 1)
    upper = (jrow < ecol).astype(jnp.float32)
    starts = jnp.dot(rcf, upper, preferred_element_type=jnp.float32)
    tcount = jnp.sum(rcf)

    tpos = (lax.broadcasted_iota(jnp.int32, (128, E), 0) * TILE).astype(
        jnp.float32)
    q = jnp.minimum(tpos, tcount - 1.0)
    ge = (starts <= q).astype(jnp.int32)
    g_ref[...] = jnp.sum(ge, axis=1, keepdims=True) - 1
    v_ref[...] = (tpos[:, :1] < tcount).astype(jnp.int32)

    nv = (tcount / TILE).astype(jnp.int32)
    l = lax.broadcasted_iota(jnp.int32, (128, 1), 0)
    ti = l // NFF
    tj = l % NFF
    serp = jnp.where(ti % 2 == 0, tj, NFF - 1 - tj)
    last_chunk = jnp.where((nv - 1) % 2 == 0, NFF - 1, 0)
    jm = jnp.where(ti < nv, serp, last_chunk)
    jm_ref[...] = jnp.where(l == 127, nv - 1, jm)

    loss_ref[...] = jnp.sum(counts, axis=1, keepdims=True) * (0.01 / NT)

    rr = lax.broadcasted_iota(jnp.int32, (128, 128), 0)
    cc = lax.broadcasted_iota(jnp.int32, (128, 128), 1)
    ltri = (cc < rr).astype(jnp.float32)

    def chunk(k, carry):
        ohc = oh_ref[pl.ds(k * 128, 128), :]
        rank = jnp.dot(ltri, ohc, preferred_element_type=jnp.float32) + carry
        posf = jnp.sum(ohc * (starts + rank), axis=1, keepdims=True)
        pos_ref[pl.ds(k * 128, 128), :] = posf.astype(jnp.int32)
        return carry + jnp.sum(ohc, axis=0, keepdims=True)

    lax.fori_loop(0, NT // 128, chunk, jnp.zeros((1, E), jnp.float32))


_gate_call = pl.pallas_call(
    _gate_body,
    out_shape=(
        jax.ShapeDtypeStruct((NT, 1), jnp.int32),
        jax.ShapeDtypeStruct((NT, 1), jnp.float32),
        jax.ShapeDtypeStruct((128, 1), jnp.int32),
        jax.ShapeDtypeStruct((128, 1), jnp.int32),
        jax.ShapeDtypeStruct((128, 1), jnp.int32),
        jax.ShapeDtypeStruct((1, 1), jnp.float32),
    ),
    scratch_shapes=[pltpu.VMEM((NT, E), jnp.float32)],
)


@functools.cache
def _make_dispatch():
    mesh = plsc.VectorSubcoreMesh(core_axis_name="c", subcore_axis_name="s")

    H = TPW // 2

    @functools.partial(
        pl.kernel,
        out_type=(
            jax.ShapeDtypeStruct((P, D), jnp.float32),
            jax.ShapeDtypeStruct((P, 128), jnp.float32),
        ),
        mesh=mesh,
        scratch_types=[
            pltpu.VMEM((H,), jnp.int32),
            pltpu.VMEM((H,), jnp.int32),
            pltpu.VMEM((TPW,), jnp.int32),
            pltpu.VMEM((H, D), jnp.float32),
            pltpu.VMEM((H, D), jnp.float32),
            pltpu.VMEM((TPW,), jnp.float32),
            pltpu.VMEM((TPW, 128), jnp.float32),
            pltpu.SemaphoreType.DMA,
            pltpu.SemaphoreType.DMA,
            pltpu.SemaphoreType.DMA,
            pltpu.SemaphoreType.DMA,
            pltpu.SemaphoreType.DMA,
        ],
    )
    def dispatch(x_hbm, pos_hbm, mv_hbm, xs_hbm, mv2_hbm,
                 pos_a, pos_b, pos_v, rows_a, rows_b, mv_v, mvr_v,
                 s1, s2, s3, s4, s5):
        wid = lax.axis_index("s") * 2 + lax.axis_index("c")
        base = wid * TPW
        in_a = pltpu.async_copy(x_hbm.at[pl.ds(base, H)], rows_a, s1)
        in_b = pltpu.async_copy(x_hbm.at[pl.ds(base + H, H)], rows_b, s2)
        pltpu.sync_copy(pos_hbm.at[pl.ds(base, H)], pos_a)
        pltpu.sync_copy(pos_hbm.at[pl.ds(base + H, H)], pos_b)
        pltpu.sync_copy(pos_hbm.at[pl.ds(base, TPW)], pos_v)
        pltpu.sync_copy(mv_hbm.at[pl.ds(base, TPW)], mv_v)
        for c in range(TPW // 16):
            v = mv_v[pl.ds(c * 16, 16)]
            for l in range(16):
                mvr_v[c * 16 + l, pl.ds(0, 16)] = jnp.full((16,), v[l],
                                                           jnp.float32)
        cp_mv = pltpu.async_copy(mvr_v, mv2_hbm.at[pos_v], s5)
        in_a.wait()
        out_a = pltpu.async_copy(rows_a, xs_hbm.at[pos_a], s3)
        in_b.wait()
        out_b = pltpu.async_copy(rows_b, xs_hbm.at[pos_b], s4)
        out_a.wait()
        out_b.wait()
        cp_mv.wait()

    return dispatch


def _ffn_body(g_s, v_s, jm_s, xs_ref, w1_ref, b1_ref, w2_ref, b2_ref,
              mvt_ref, out_ref):
    i = pl.program_id(0)
    j = pl.program_id(1)

    @pl.when(v_s[i] == 1)
    def _():
        xb = xs_ref[...].astype(jnp.bfloat16)
        h = jnp.dot(xb, w1_ref[0].astype(jnp.bfloat16),
                    preferred_element_type=jnp.float32)
        hb = jnp.maximum(h + b1_ref[0], 0.0).astype(jnp.bfloat16)
        part = jnp.dot(hb, w2_ref[0].astype(jnp.bfloat16),
                       preferred_element_type=jnp.float32)

        @pl.when(j == 0)
        def _():
            out_ref[...] = part

        @pl.when(jnp.logical_and(j > 0, j < NFF - 1))
        def _():
            out_ref[...] += part

        @pl.when(j == NFF - 1)
        def _():
            out_ref[...] = (out_ref[...] + part + b2_ref[0]) \
                * mvt_ref[...][:, :1]


def _ti(i, v, jm):
    return jnp.where(v[i] == 1, i, jm[127])


_ffn_call = pl.pallas_call(
    _ffn_body,
    grid_spec=pltpu.PrefetchScalarGridSpec(
        num_scalar_prefetch=3,
        grid=(T, NFF),
        in_specs=[
            pl.BlockSpec((TILE, D), lambda i, j, g, v, jm: (_ti(i, v, jm), 0)),
            pl.BlockSpec((1, D, FB),
                         lambda i, j, g, v, jm: (g[i], 0, jm[NFF * i + j])),
            pl.BlockSpec((1, 1, FB),
                         lambda i, j, g, v, jm: (g[i], 0, jm[NFF * i + j])),
            pl.BlockSpec((1, FB, D),
                         lambda i, j, g, v, jm: (g[i], jm[NFF * i + j], 0)),
            pl.BlockSpec((1, 1, D), lambda i, j, g, v, jm: (g[i], 0, 0)),
            pl.BlockSpec((TILE, 128),
                         lambda i, j, g, v, jm: (_ti(i, v, jm), 0)),
        ],
        out_specs=pl.BlockSpec((TILE, D),
                               lambda i, j, g, v, jm: (_ti(i, v, jm), 0)),
    ),
    out_shape=jax.ShapeDtypeStruct((P, D), jnp.float32),
    compiler_params=pltpu.CompilerParams(
        dimension_semantics=("arbitrary", "arbitrary")),
)


@functools.cache
def _make_combine():
    mesh = plsc.VectorSubcoreMesh(core_axis_name="c", subcore_axis_name="s")

    H = TPW // 2

    @functools.partial(
        pl.kernel,
        out_type=jax.ShapeDtypeStruct((NT, D), jnp.float32),
        mesh=mesh,
        scratch_types=[
            pltpu.VMEM((H,), jnp.int32),
            pltpu.VMEM((H,), jnp.int32),
            pltpu.VMEM((H, D), jnp.float32),
            pltpu.VMEM((H, D), jnp.float32),
            pltpu.SemaphoreType.DMA,
            pltpu.SemaphoreType.DMA,
            pltpu.SemaphoreType.DMA,
            pltpu.SemaphoreType.DMA,
        ],
    )
    def combine(ys_hbm, pos_hbm, out_hbm,
                pos_a, pos_b, rows_a, rows_b, s1, s2, s3, s4):
        wid = lax.axis_index("s") * 2 + lax.axis_index("c")
        base = wid * TPW
        pltpu.sync_copy(pos_hbm.at[pl.ds(base, H)], pos_a)
        pltpu.sync_copy(pos_hbm.at[pl.ds(base + H, H)], pos_b)
        in_a = pltpu.async_copy(ys_hbm.at[pos_a], rows_a, s1)
        in_b = pltpu.async_copy(ys_hbm.at[pos_b], rows_b, s2)
        in_a.wait()
        out_a = pltpu.async_copy(rows_a, out_hbm.at[pl.ds(base, H)], s3)
        in_b.wait()
        out_b = pltpu.async_copy(rows_b, out_hbm.at[pl.ds(base + H, H)], s4)
        out_a.wait()
        out_b.wait()

    return combine


def kernel(x, gate_w, gate_b, w1, b1, w2, b2):
    x2d = x.reshape(NT, D)
    pos2, mv2, g2, v2, jm2, loss = _gate_call(x2d, gate_w,
                                              gate_b.reshape(1, E))
    pos = pos2.reshape(NT)
    mv = mv2.reshape(NT)
    g = g2.reshape(128)[:T]
    vld = v2.reshape(128)[:T]
    jm = jm2.reshape(128)
    xs, mvt = _make_dispatch()(x2d, pos, mv)
    ys = _ffn_call(g, vld, jm, xs, w1, b1.reshape(E, 1, DFF), w2,
                   b2.reshape(E, 1, D), mvt)
    y2d = _make_combine()(ys, pos)
    return y2d.reshape(x.shape), loss.reshape(())
